# Initial kernel scaffold; baseline (speedup 1.0000x reference)
import jax
import jax.numpy as jnp
from jax.experimental import pallas as pl


def kernel(features, edge_index, etypes, norm, W1, loop1, b1, W2, loop2, b2):
    raise NotImplementedError("write your pallas kernel here")


# speedup vs baseline: 15.3582x; 15.3582x over previous
import functools

import jax
import jax.numpy as jnp
from jax import lax
from jax.experimental import pallas as pl
from jax.experimental.pallas import tpu as pltpu
from jax.experimental.pallas import tpu_sc as plsc

_NC = 2
_NS = 16
_NW = _NC * _NS
_CH = 128
_LANES = 16


def _sc_gather_scatter(n_nodes, n_hid, e_pad, hrel, gidx, dst, nrm):
    epw = e_pad // _NW
    nchunk = epw // _CH
    n_pad = ((n_nodes + _NS * 8 - 1) // (_NS * 8)) * (_NS * 8)
    rpt = n_pad // _NS
    nvec = n_hid // _LANES

    mesh = plsc.VectorSubcoreMesh(core_axis_name="c", subcore_axis_name="s",
                                  num_cores=_NC, num_subcores=_NS)

    @functools.partial(
        pl.kernel,
        out_type=jax.ShapeDtypeStruct((_NC * n_pad, n_hid), jnp.float32),
        mesh=mesh,
        scratch_types=[
            pltpu.VMEM((_CH,), jnp.int32),
            pltpu.VMEM((_CH,), jnp.int32),
            pltpu.VMEM((_CH,), jnp.float32),
            pltpu.VMEM((_CH, n_hid), jnp.float32),
            pltpu.VMEM_SHARED((n_pad, n_hid), jnp.float32),
            pltpu.SemaphoreType.DMA,
        ],
    )
    def k(hrel_hbm, gidx_hbm, dst_hbm, nrm_hbm, out_hbm,
          idx_v, dst_v, nrm_v, rows_v, agg_sh, sem):
        cid = lax.axis_index("c")
        sid = lax.axis_index("s")
        wid = cid * _NS + sid

        zeros16 = jnp.zeros((_LANES,), jnp.float32)

        def zero_row(i, carry):
            for j in range(nvec):
                rows_v[i, pl.ds(j * _LANES, _LANES)] = zeros16
            return carry

        lax.fori_loop(0, _CH, zero_row, 0)
        row0 = sid * rpt
        done = 0
        while done < rpt:
            sz = min(_CH, rpt - done)
            pltpu.sync_copy(rows_v.at[pl.ds(0, sz)],
                            agg_sh.at[pl.ds(row0 + done, sz)])
            done += sz
        plsc.subcore_barrier()

        base = wid * epw

        def chunk(t, carry):
            off = base + t * _CH
            pltpu.sync_copy(gidx_hbm.at[pl.ds(off, _CH)], idx_v)
            pltpu.sync_copy(dst_hbm.at[pl.ds(off, _CH)], dst_v)
            pltpu.sync_copy(nrm_hbm.at[pl.ds(off, _CH)], nrm_v)
            pltpu.async_copy(hrel_hbm.at[idx_v], rows_v, sem).wait()

            def group(g, c2):
                norms = nrm_v[pl.ds(g * _LANES, _LANES)]
                e0 = g * _LANES
                for lane in range(_LANES):
                    s = norms[lane]
                    for j in range(nvec):
                        sl = pl.ds(j * _LANES, _LANES)
                        rows_v[e0 + lane, sl] = rows_v[e0 + lane, sl] * s
                return c2

            lax.fori_loop(0, _CH // _LANES, group, 0)
            pltpu.sync_copy(rows_v, agg_sh.at[dst_v], add=True)
            return carry

        lax.fori_loop(0, nchunk, chunk, 0)

        plsc.subcore_barrier()
        done = 0
        while done < rpt:
            sz = min(_CH, rpt - done)
            pltpu.sync_copy(agg_sh.at[pl.ds(row0 + done, sz)],
                            out_hbm.at[pl.ds(cid * n_pad + row0 + done, sz)])
            done += sz

    out = k(hrel, gidx, dst, nrm)
    return out.reshape(_NC, n_pad, n_hid)[:, :n_nodes, :]


def _matmul(x, ws):
    n, d = x.shape
    r1, _, h = ws.shape
    bn = 2000
    nb = n // bn

    def body(x_ref, w_ref, o_ref):
        o_ref[0] = jnp.dot(x_ref[...], w_ref[0],
                           preferred_element_type=jnp.float32)

    return pl.pallas_call(
        body,
        grid=(nb, r1),
        in_specs=[pl.BlockSpec((bn, d), lambda i, r: (i, 0)),
                  pl.BlockSpec((1, d, h), lambda i, r: (r, 0, 0))],
        out_specs=pl.BlockSpec((1, bn, h), lambda i, r: (r, i, 0)),
        out_shape=jax.ShapeDtypeStruct((r1, n, h), jnp.float32),
    )(x, ws)


def _combine_matmul(parts, hall, b, ws):
    n, hdim = parts.shape[1], parts.shape[2]
    r1, _, out_dim = ws.shape
    loop_row = hall.shape[0] - 1
    bn = 2000
    nb = n // bn

    def body(p_ref, lt_ref, b_ref, w_ref, o_ref):
        hblk = p_ref[0] + p_ref[1] + lt_ref[0] + b_ref[0]
        hblk = jnp.maximum(hblk, 0.0)
        o_ref[0] = jnp.dot(hblk, w_ref[0], preferred_element_type=jnp.float32)

    return pl.pallas_call(
        body,
        grid=(nb, r1),
        in_specs=[pl.BlockSpec((2, bn, hdim), lambda i, r: (0, i, 0)),
                  pl.BlockSpec((1, bn, hdim), lambda i, r: (loop_row, i, 0)),
                  pl.BlockSpec((1, hdim), lambda i, r: (0, 0)),
                  pl.BlockSpec((1, hdim, out_dim), lambda i, r: (r, 0, 0))],
        out_specs=pl.BlockSpec((1, bn, out_dim), lambda i, r: (r, i, 0)),
        out_shape=jax.ShapeDtypeStruct((r1, n, out_dim), jnp.float32),
    )(parts, hall, b.reshape(1, -1), ws)


def _final_sum(parts, hall, b):
    n, d = parts.shape[1], parts.shape[2]
    loop_row = hall.shape[0] - 1
    bn = 2000
    nb = n // bn

    def body(p_ref, lt_ref, b_ref, o_ref):
        o_ref[...] = p_ref[0] + p_ref[1] + lt_ref[0] + b_ref[0]

    return pl.pallas_call(
        body,
        grid=(nb,),
        in_specs=[pl.BlockSpec((2, bn, d), lambda i: (0, i, 0)),
                  pl.BlockSpec((1, bn, d), lambda i: (loop_row, i, 0)),
                  pl.BlockSpec((1, d), lambda i: (0, 0))],
        out_specs=pl.BlockSpec((bn, d), lambda i: (i, 0)),
        out_shape=jax.ShapeDtypeStruct((n, d), jnp.float32),
    )(parts, hall, b.reshape(1, -1))


def kernel(features, edge_index, etypes, norm, W1, loop1, b1, W2, loop2, b2):
    n, _ = features.shape
    e = etypes.shape[0]
    src = edge_index[0].astype(jnp.int32)
    dstv = edge_index[1].astype(jnp.int32)
    et = etypes.astype(jnp.int32)
    gidx = et * n + src
    nrm = norm[:, 0]

    granule = _NW * _CH
    e_pad = ((e + granule - 1) // granule) * granule
    pad = e_pad - e
    if pad:
        spread = jnp.arange(pad, dtype=jnp.int32) % n
        gidx = jnp.concatenate([gidx, spread])
        dstv = jnp.concatenate([dstv, spread])
        nrm = jnp.concatenate([nrm, jnp.zeros((pad,), jnp.float32)])

    ws1 = jnp.concatenate([W1, loop1[None]], axis=0)
    ws2 = jnp.concatenate([W2, loop2[None]], axis=0)

    hall1 = _matmul(features, ws1)
    p1 = _sc_gather_scatter(n, hall1.shape[2], e_pad,
                            hall1.reshape(-1, hall1.shape[2]), gidx, dstv, nrm)
    hall2 = _combine_matmul(p1, hall1, b1, ws2)
    p2 = _sc_gather_scatter(n, hall2.shape[2], e_pad,
                            hall2.reshape(-1, hall2.shape[2]), gidx, dstv, nrm)
    return _final_sum(p2, hall2, b2)


# speedup vs baseline: 28.5678x; 1.8601x over previous
import functools

import jax
import jax.numpy as jnp
from jax import lax
from jax.experimental import pallas as pl
from jax.experimental.pallas import tpu as pltpu
from jax.experimental.pallas import tpu_sc as plsc

_NC = 2
_NS = 16
_NW = _NC * _NS
_CH = 128
_NSEG = 4
_LANES = 16


def _sc_gather_scatter(n_nodes, n_hid, e_pad, hrel, gidx, dst, nrm):
    epw = e_pad // _NW
    nchunk = epw // _CH
    n_pad = ((n_nodes + _NS * 8 - 1) // (_NS * 8)) * (_NS * 8)
    rpt = n_pad // _NS
    nvec = n_hid // _LANES

    mesh = plsc.VectorSubcoreMesh(core_axis_name="c", subcore_axis_name="s",
                                  num_cores=_NC, num_subcores=_NS)

    nseg = _NSEG
    cps = nchunk // nseg

    @functools.partial(
        pl.kernel,
        out_type=jax.ShapeDtypeStruct((_NC * n_pad, n_hid), jnp.float32),
        mesh=mesh,
        scratch_types=[
            pltpu.VMEM((cps, _CH), jnp.int32),
            pltpu.VMEM((cps, _CH), jnp.int32),
            pltpu.VMEM((cps * _CH,), jnp.float32),
            pltpu.VMEM((_CH, n_hid), jnp.float32),
            pltpu.VMEM((_CH, n_hid), jnp.float32),
            pltpu.VMEM_SHARED((n_pad, n_hid), jnp.float32),
            pltpu.SemaphoreType.DMA,
            pltpu.SemaphoreType.DMA,
        ],
    )
    def k(hrel_hbm, gidx_hbm, dst_hbm, nrm_hbm, out_hbm,
          idx_v, dst_v, nrm_v, rows0, rows1, agg_sh, sem0, sem1):
        cid = lax.axis_index("c")
        sid = lax.axis_index("s")
        wid = cid * _NS + sid

        zeros16 = jnp.zeros((_LANES,), jnp.float32)

        def zero_row(i, carry):
            for j in range(nvec):
                rows0[i, pl.ds(j * _LANES, _LANES)] = zeros16
            return carry

        lax.fori_loop(0, _CH, zero_row, 0)
        row0 = sid * rpt
        done = 0
        while done < rpt:
            sz = min(_CH, rpt - done)
            pltpu.sync_copy(rows0.at[pl.ds(0, sz)],
                            agg_sh.at[pl.ds(row0 + done, sz)])
            done += sz
        plsc.subcore_barrier()

        def scale(t, buf):
            def group(g, c2):
                norms = nrm_v[pl.ds(t * _CH + g * _LANES, _LANES)]
                e0 = g * _LANES
                for lane in range(_LANES):
                    s = norms[lane]
                    for j in range(nvec):
                        sl = pl.ds(j * _LANES, _LANES)
                        buf[e0 + lane, sl] = buf[e0 + lane, sl] * s
                return c2

            lax.fori_loop(0, _CH // _LANES, group, 0)

        def seg(s2, carry):
            pltpu.sync_copy(gidx_hbm.at[wid, s2], idx_v)
            pltpu.sync_copy(dst_hbm.at[wid, s2], dst_v)
            pltpu.sync_copy(nrm_hbm.at[wid, s2], nrm_v)
            pltpu.async_copy(hrel_hbm.at[idx_v.at[0]], rows0, sem0)

            def pair(p, c2):
                t0 = 2 * p
                t1 = t0 + 1
                pltpu.async_copy(hrel_hbm.at[idx_v.at[t1]], rows1, sem1)
                pltpu.make_async_copy(hrel_hbm.at[idx_v.at[t0]], rows0,
                                      sem0).wait()
                scale(t0, rows0)
                pltpu.sync_copy(rows0, agg_sh.at[dst_v.at[t0]], add=True)

                @pl.when(t1 + 1 < cps)
                def _():
                    pltpu.async_copy(hrel_hbm.at[idx_v.at[t1 + 1]], rows0, sem0)

                pltpu.make_async_copy(hrel_hbm.at[idx_v.at[t1]], rows1,
                                      sem1).wait()
                scale(t1, rows1)
                pltpu.sync_copy(rows1, agg_sh.at[dst_v.at[t1]], add=True)
                return c2

            lax.fori_loop(0, cps // 2, pair, 0)
            return carry

        lax.fori_loop(0, nseg, seg, 0)

        plsc.subcore_barrier()
        done = 0
        while done < rpt:
            sz = min(_CH, rpt - done)
            pltpu.sync_copy(agg_sh.at[pl.ds(row0 + done, sz)],
                            out_hbm.at[pl.ds(cid * n_pad + row0 + done, sz)])
            done += sz

    out = k(hrel,
            gidx.reshape(_NW, nseg, cps, _CH),
            dst.reshape(_NW, nseg, cps, _CH),
            nrm.reshape(_NW, nseg, cps * _CH))
    return out.reshape(_NC, n_pad, n_hid)[:, :n_nodes, :]


def _matmul(x, ws):
    n, d = x.shape
    r1, _, h = ws.shape
    bn = 2000
    nb = n // bn

    def body(x_ref, w_ref, o_ref):
        o_ref[0] = jnp.dot(x_ref[...], w_ref[0],
                           preferred_element_type=jnp.float32)

    return pl.pallas_call(
        body,
        grid=(nb, r1),
        in_specs=[pl.BlockSpec((bn, d), lambda i, r: (i, 0)),
                  pl.BlockSpec((1, d, h), lambda i, r: (r, 0, 0))],
        out_specs=pl.BlockSpec((1, bn, h), lambda i, r: (r, i, 0)),
        out_shape=jax.ShapeDtypeStruct((r1, n, h), jnp.float32),
    )(x, ws)


def _combine_matmul(parts, hall, b, ws):
    n, hdim = parts.shape[1], parts.shape[2]
    r1, _, out_dim = ws.shape
    loop_row = hall.shape[0] - 1
    bn = 2000
    nb = n // bn

    def body(p_ref, lt_ref, b_ref, w_ref, o_ref):
        hblk = p_ref[0] + p_ref[1] + lt_ref[0] + b_ref[0]
        hblk = jnp.maximum(hblk, 0.0)
        o_ref[0] = jnp.dot(hblk, w_ref[0], preferred_element_type=jnp.float32)

    return pl.pallas_call(
        body,
        grid=(nb, r1),
        in_specs=[pl.BlockSpec((2, bn, hdim), lambda i, r: (0, i, 0)),
                  pl.BlockSpec((1, bn, hdim), lambda i, r: (loop_row, i, 0)),
                  pl.BlockSpec((1, hdim), lambda i, r: (0, 0)),
                  pl.BlockSpec((1, hdim, out_dim), lambda i, r: (r, 0, 0))],
        out_specs=pl.BlockSpec((1, bn, out_dim), lambda i, r: (r, i, 0)),
        out_shape=jax.ShapeDtypeStruct((r1, n, out_dim), jnp.float32),
    )(parts, hall, b.reshape(1, -1), ws)


def _final_sum(parts, hall, b):
    n, d = parts.shape[1], parts.shape[2]
    loop_row = hall.shape[0] - 1
    bn = 2000
    nb = n // bn

    def body(p_ref, lt_ref, b_ref, o_ref):
        o_ref[...] = p_ref[0] + p_ref[1] + lt_ref[0] + b_ref[0]

    return pl.pallas_call(
        body,
        grid=(nb,),
        in_specs=[pl.BlockSpec((2, bn, d), lambda i: (0, i, 0)),
                  pl.BlockSpec((1, bn, d), lambda i: (loop_row, i, 0)),
                  pl.BlockSpec((1, d), lambda i: (0, 0))],
        out_specs=pl.BlockSpec((bn, d), lambda i: (i, 0)),
        out_shape=jax.ShapeDtypeStruct((n, d), jnp.float32),
    )(parts, hall, b.reshape(1, -1))


def kernel(features, edge_index, etypes, norm, W1, loop1, b1, W2, loop2, b2):
    n, _ = features.shape
    e = etypes.shape[0]
    src = edge_index[0].astype(jnp.int32)
    dstv = edge_index[1].astype(jnp.int32)
    et = etypes.astype(jnp.int32)
    gidx = et * n + src
    nrm = norm[:, 0]

    granule = _NW * _CH * _NSEG
    e_pad = ((e + granule - 1) // granule) * granule
    pad = e_pad - e
    if pad:
        spread = jnp.arange(pad, dtype=jnp.int32) % n
        gidx = jnp.concatenate([gidx, spread])
        dstv = jnp.concatenate([dstv, spread])
        nrm = jnp.concatenate([nrm, jnp.zeros((pad,), jnp.float32)])

    ws1 = jnp.concatenate([W1, loop1[None]], axis=0)
    ws2 = jnp.concatenate([W2, loop2[None]], axis=0)

    hall1 = _matmul(features, ws1)
    p1 = _sc_gather_scatter(n, hall1.shape[2], e_pad,
                            hall1.reshape(-1, hall1.shape[2]), gidx, dstv, nrm)
    hall2 = _combine_matmul(p1, hall1, b1, ws2)
    p2 = _sc_gather_scatter(n, hall2.shape[2], e_pad,
                            hall2.reshape(-1, hall2.shape[2]), gidx, dstv, nrm)
    return _final_sum(p2, hall2, b2)
